# Initial kernel scaffold; baseline (speedup 1.0000x reference)
import jax
import jax.numpy as jnp
from jax.experimental import pallas as pl


def kernel(pred_logits, pred_boxes, target_sizes, positive_map):
    raise NotImplementedError("write your pallas kernel here")


# speedup vs baseline: 1.2137x; 1.2137x over previous
import functools

import jax
import jax.numpy as jnp
from jax import lax
from jax.experimental import pallas as pl
from jax.experimental.pallas import tpu as pltpu

NSEL = 300
CAND = 512
_B, _Q, _T, _C = 128, 900, 256, 25
_F = 4
_R = _Q // _F
_M = _F * _C
_LANES = 128
_KIN = _F * _T


def _body(lg_ref, box_ref, ts_ref, pm_ref, sc_ref, lb_ref, bx_ref):
    x = lg_ref[0]
    s = jax.nn.sigmoid(x)
    y = lax.dot_general(s, pm_ref[...], (((1,), (0,)), ((), ())),
                        preferred_element_type=jnp.float32)
    kbits = lax.bitcast_convert_type(y, jnp.int32)

    def bs(_, lohi):
        lo, hi = lohi
        mid = (lo + hi) // 2
        cnt = jnp.sum((kbits >= mid).astype(jnp.int32))
        big = cnt >= NSEL
        return jnp.where(big, mid, lo), jnp.where(big, hi, mid)

    lo, _ = lax.fori_loop(0, 30, bs, (jnp.int32(1), jnp.int32(1 << 30)))
    tstar = lo

    mask = (kbits >= tstar).astype(jnp.int32)

    rc = jnp.sum(mask, axis=1, keepdims=True)
    cum = rc
    sh = 1
    while sh < _R:
        cum = cum + jnp.concatenate(
            [jnp.zeros((sh, 1), jnp.int32), cum[:-sh, :]], axis=0)
        sh *= 2
    total = cum[_R - 1, 0]

    cl = mask
    sh = 1
    while sh < _LANES:
        cl = cl + jnp.concatenate(
            [jnp.zeros((_R, sh), jnp.int32), cl[:, :-sh]], axis=1)
        sh *= 2

    p_col = lax.broadcasted_iota(jnp.int32, (CAND, 1), 0)
    cum_row = jnp.transpose(cum)
    rc_row = jnp.transpose(rc)
    a = (jnp.broadcast_to(cum_row, (CAND, _R))
         <= lax.broadcasted_iota(jnp.int32, (CAND, _R), 0)).astype(jnp.int32)
    r_p = jnp.sum(a, axis=1, keepdims=True)
    ro_p = jnp.sum(a * rc_row, axis=1, keepdims=True)
    w_p = p_col - ro_p
    r_pc = jnp.minimum(r_p, _R - 1)
    onehot_r = (lax.broadcasted_iota(jnp.int32, (CAND, _R), 1)
                == r_pc).astype(jnp.float32)
    clg = lax.dot_general(onehot_r, cl.astype(jnp.float32),
                          (((1,), (0,)), ((), ())),
                          preferred_element_type=jnp.float32)
    yg = lax.dot_general(onehot_r, y, (((1,), (0,)), ((), ())),
                         preferred_element_type=jnp.float32, precision=lax.Precision.HIGHEST)
    l_p = jnp.sum((clg <= w_p.astype(jnp.float32)).astype(jnp.int32),
                  axis=1, keepdims=True)
    l_pc = jnp.minimum(l_p, _LANES - 1)
    lmask = (lax.broadcasted_iota(jnp.int32, (CAND, _LANES), 1) == l_pc)
    val = jnp.sum(jnp.where(lmask, yg, 0.0), axis=1, keepdims=True)

    flat = r_pc * _M + l_pc
    valid = p_col < total
    key = lax.bitcast_convert_type(val, jnp.int32)
    key = jnp.where(valid, key, -1)
    flat_tb = jnp.where(valid, flat, (1 << 29) + p_col)
    q_p = flat // _C
    c_p = flat - q_p * _C

    key_row = jnp.transpose(key)
    flat_row = jnp.transpose(flat_tb)
    prec = ((key_row > key) |
            ((key_row == key) & (flat_row < flat_tb))).astype(jnp.int32)
    rank = jnp.sum(prec, axis=1, keepdims=True)

    rank_row = jnp.transpose(rank)
    j_col = lax.broadcasted_iota(jnp.int32, (CAND, CAND), 0)
    perm = (jnp.broadcast_to(rank_row, (CAND, CAND))
            == j_col).astype(jnp.float32)

    b = box_ref[0]
    cx = b[:, 0:1]
    cy = b[:, 1:2]
    w = b[:, 2:3]
    h = b[:, 3:4]
    xyxy = jnp.concatenate(
        [cx - 0.5 * w, cy - 0.5 * h, cx + 0.5 * w, cy + 0.5 * h], axis=1)
    onehot_q = (lax.broadcasted_iota(jnp.int32, (CAND, _Q), 1)
                == q_p).astype(jnp.float32)
    bq = lax.dot_general(onehot_q, xyxy, (((1,), (0,)), ((), ())),
                         preferred_element_type=jnp.float32, precision=lax.Precision.HIGHEST)

    fields = jnp.concatenate([val, c_p.astype(jnp.float32), bq], axis=1)
    sorted_f = lax.dot_general(perm, fields, (((1,), (0,)), ((), ())),
                               preferred_element_type=jnp.float32, precision=lax.Precision.HIGHEST)
    sc_ref[0] = sorted_f[:NSEL, 0:1]
    lb_ref[0] = sorted_f[:NSEL, 1:2].astype(jnp.int32)

    img_h = ts_ref[0, 0, 0].astype(jnp.float32)
    img_w = ts_ref[0, 0, 1].astype(jnp.float32)
    li = lax.broadcasted_iota(jnp.int32, (1, 4), 1)
    scale = jnp.where(li % 2 == 0, img_w, img_h)
    bx_ref[0] = sorted_f[:NSEL, 2:6] * scale


@jax.jit
def kernel(pred_logits, pred_boxes, target_sizes, positive_map):
    lg2 = pred_logits.reshape(_B, _R, _KIN)
    pm_t = positive_map.T
    pm_big = jnp.zeros((_KIN, _LANES), jnp.float32)
    for j in range(_F):
        pm_big = pm_big.at[j * _T:(j + 1) * _T, j * _C:(j + 1) * _C].set(pm_t)
    ts3 = target_sizes.reshape(_B, 1, 2)

    grid = (_B,)
    scores3, labels3, boxes = pl.pallas_call(
        _body,
        grid=grid,
        in_specs=[
            pl.BlockSpec((1, _R, _KIN), lambda b: (b, 0, 0)),
            pl.BlockSpec((1, _Q, 4), lambda b: (b, 0, 0)),
            pl.BlockSpec((1, 1, 2), lambda b: (b, 0, 0),
                         memory_space=pltpu.SMEM),
            pl.BlockSpec((_KIN, _LANES), lambda b: (0, 0)),
        ],
        out_specs=[
            pl.BlockSpec((1, NSEL, 1), lambda b: (b, 0, 0)),
            pl.BlockSpec((1, NSEL, 1), lambda b: (b, 0, 0)),
            pl.BlockSpec((1, NSEL, 4), lambda b: (b, 0, 0)),
        ],
        out_shape=[
            jax.ShapeDtypeStruct((_B, NSEL, 1), jnp.float32),
            jax.ShapeDtypeStruct((_B, NSEL, 1), jnp.int32),
            jax.ShapeDtypeStruct((_B, NSEL, 4), jnp.float32),
        ],
    )(lg2, pred_boxes, ts3, pm_big)
    return scores3[..., 0], labels3[..., 0], boxes


# speedup vs baseline: 1.3975x; 1.1514x over previous
import functools

import jax
import jax.numpy as jnp
from jax import lax
from jax.experimental import pallas as pl
from jax.experimental.pallas import tpu as pltpu

NSEL = 300
CAND = 384
_B, _Q, _T, _C = 128, 900, 256, 25
_F = 4
_R = _Q // _F
_M = _F * _C
_LANES = 128
_KIN = _F * _T


def _body(lg_ref, box_ref, ts_ref, pm_ref, sc_ref, lb_ref, bx_ref):
    x = lg_ref[0]
    s = jax.nn.sigmoid(x)
    y = lax.dot_general(s, pm_ref[...], (((1,), (0,)), ((), ())),
                        preferred_element_type=jnp.float32)
    kbits = lax.bitcast_convert_type(y, jnp.int32)

    def bs(_, lohi):
        lo, hi = lohi
        mid = (lo + hi) // 2
        cnt = jnp.sum((kbits >= mid).astype(jnp.int32))
        big = cnt >= NSEL
        return jnp.where(big, mid, lo), jnp.where(big, hi, mid)

    lo, _ = lax.fori_loop(0, 30, bs, (jnp.int32(1), jnp.int32(1 << 30)))
    tstar = lo

    mask = (kbits >= tstar).astype(jnp.int32)

    rc = jnp.sum(mask, axis=1, keepdims=True)
    cum = rc
    sh = 1
    while sh < _R:
        cum = cum + jnp.concatenate(
            [jnp.zeros((sh, 1), jnp.int32), cum[:-sh, :]], axis=0)
        sh *= 2
    total = cum[_R - 1, 0]

    cl = mask
    sh = 1
    while sh < _LANES:
        cl = cl + jnp.concatenate(
            [jnp.zeros((_R, sh), jnp.int32), cl[:, :-sh]], axis=1)
        sh *= 2

    p_col = lax.broadcasted_iota(jnp.int32, (CAND, 1), 0)
    cum_row = jnp.transpose(cum)
    a = (jnp.broadcast_to(cum_row, (CAND, _R))
         <= lax.broadcasted_iota(jnp.int32, (CAND, _R), 0)).astype(jnp.float32)
    ones_rc = jnp.concatenate(
        [jnp.ones((_R, 1), jnp.float32), rc.astype(jnp.float32)],
        axis=1)
    rr = lax.dot_general(a, ones_rc, (((1,), (0,)), ((), ())),
                         preferred_element_type=jnp.float32)
    r_p = rr[:, 0:1].astype(jnp.int32)
    ro_p = rr[:, 1:2].astype(jnp.int32)
    w_p = p_col - ro_p
    r_pc = jnp.minimum(r_p, _R - 1)
    onehot_r = (lax.broadcasted_iota(jnp.int32, (CAND, _R), 1)
                == r_pc).astype(jnp.float32)
    clg = lax.dot_general(onehot_r, cl.astype(jnp.float32),
                          (((1,), (0,)), ((), ())),
                          preferred_element_type=jnp.float32)
    yg = lax.dot_general(onehot_r, y, (((1,), (0,)), ((), ())),
                         preferred_element_type=jnp.float32,
                         precision=lax.Precision.HIGHEST)
    lcmp = (clg <= w_p.astype(jnp.float32)).astype(jnp.float32)
    l_p = lax.dot_general(lcmp, jnp.ones((_LANES, 1), jnp.float32),
                          (((1,), (0,)), ((), ())),
                          preferred_element_type=jnp.float32).astype(jnp.int32)
    l_pc = jnp.minimum(l_p, _LANES - 1)
    lmask = (lax.broadcasted_iota(jnp.int32, (CAND, _LANES), 1) == l_pc)
    val = jnp.sum(jnp.where(lmask, yg, 0.0), axis=1, keepdims=True)

    flat = r_pc * _M + l_pc
    valid = p_col < total
    key = lax.bitcast_convert_type(val, jnp.int32)
    key = jnp.where(valid, key, -1)
    flat_tb = jnp.where(valid, flat, (1 << 29) + p_col)
    q_p = flat // _C
    c_p = flat - q_p * _C

    key_row = jnp.transpose(key)
    flat_row = jnp.transpose(flat_tb)
    prec = ((key_row > key) |
            ((key_row == key) & (flat_row < flat_tb))).astype(jnp.float32)
    rank = lax.dot_general(prec, jnp.ones((CAND, 1), jnp.float32),
                           (((1,), (0,)), ((), ())),
                           preferred_element_type=jnp.float32).astype(jnp.int32)

    rank_row = jnp.transpose(rank)
    j_col = lax.broadcasted_iota(jnp.int32, (NSEL, CAND), 0)
    perm = (jnp.broadcast_to(rank_row, (NSEL, CAND))
            == j_col).astype(jnp.float32)

    b = box_ref[0]
    cx = b[:, 0:1]
    cy = b[:, 1:2]
    w = b[:, 2:3]
    h = b[:, 3:4]
    xyxy = jnp.concatenate(
        [cx - 0.5 * w, cy - 0.5 * h, cx + 0.5 * w, cy + 0.5 * h], axis=1)
    onehot_q = (lax.broadcasted_iota(jnp.int32, (CAND, _Q), 1)
                == q_p).astype(jnp.float32)
    bq = lax.dot_general(onehot_q, xyxy, (((1,), (0,)), ((), ())),
                         preferred_element_type=jnp.float32,
                         precision=lax.Precision.HIGHEST)

    fields = jnp.concatenate([val, c_p.astype(jnp.float32), bq], axis=1)
    sorted_f = lax.dot_general(perm, fields, (((1,), (0,)), ((), ())),
                               preferred_element_type=jnp.float32,
                               precision=lax.Precision.HIGHEST)
    sc_ref[0] = sorted_f[:, 0:1]
    lb_ref[0] = sorted_f[:, 1:2].astype(jnp.int32)

    img_h = ts_ref[0, 0, 0].astype(jnp.float32)
    img_w = ts_ref[0, 0, 1].astype(jnp.float32)
    li = lax.broadcasted_iota(jnp.int32, (1, 4), 1)
    scale = jnp.where(li % 2 == 0, img_w, img_h)
    bx_ref[0] = sorted_f[:, 2:6] * scale


@jax.jit
def kernel(pred_logits, pred_boxes, target_sizes, positive_map):
    lg2 = pred_logits.reshape(_B, _R, _KIN)
    pm_t = positive_map.T
    pm_big = jnp.zeros((_KIN, _LANES), jnp.float32)
    for j in range(_F):
        pm_big = pm_big.at[j * _T:(j + 1) * _T, j * _C:(j + 1) * _C].set(pm_t)
    ts3 = target_sizes.reshape(_B, 1, 2)

    grid = (_B,)
    scores3, labels3, boxes = pl.pallas_call(
        _body,
        grid=grid,
        in_specs=[
            pl.BlockSpec((1, _R, _KIN), lambda b: (b, 0, 0)),
            pl.BlockSpec((1, _Q, 4), lambda b: (b, 0, 0)),
            pl.BlockSpec((1, 1, 2), lambda b: (b, 0, 0),
                         memory_space=pltpu.SMEM),
            pl.BlockSpec((_KIN, _LANES), lambda b: (0, 0)),
        ],
        out_specs=[
            pl.BlockSpec((1, NSEL, 1), lambda b: (b, 0, 0)),
            pl.BlockSpec((1, NSEL, 1), lambda b: (b, 0, 0)),
            pl.BlockSpec((1, NSEL, 4), lambda b: (b, 0, 0)),
        ],
        out_shape=[
            jax.ShapeDtypeStruct((_B, NSEL, 1), jnp.float32),
            jax.ShapeDtypeStruct((_B, NSEL, 1), jnp.int32),
            jax.ShapeDtypeStruct((_B, NSEL, 4), jnp.float32),
        ],
    )(lg2, pred_boxes, ts3, pm_big)
    return scores3[..., 0], labels3[..., 0], boxes


# speedup vs baseline: 2.2984x; 1.6447x over previous
import functools

import jax
import jax.numpy as jnp
from jax import lax
from jax.experimental import pallas as pl
from jax.experimental.pallas import tpu as pltpu

NSEL = 300
CAND = 384
IMGS = 2
_B, _Q, _T, _C = 128, 900, 256, 25
_F = 4
_R = _Q // _F
_M = _F * _C
_LANES = 128
_KIN = _F * _T


def _split3(v):
    m = jnp.int32(-65536)
    b1 = lax.bitcast_convert_type(
        lax.bitcast_convert_type(v, jnp.int32) & m, jnp.float32)
    r1 = v - b1
    b2 = lax.bitcast_convert_type(
        lax.bitcast_convert_type(r1, jnp.int32) & m, jnp.float32)
    b3 = r1 - b2
    return b1, b2, b3


def _dot_exact(a, v):
    b1, b2, b3 = _split3(v)
    dn = (((1,), (0,)), ((), ()))
    d1 = lax.dot_general(a, b1, dn, preferred_element_type=jnp.float32)
    d2 = lax.dot_general(a, b2, dn, preferred_element_type=jnp.float32)
    d3 = lax.dot_general(a, b3, dn, preferred_element_type=jnp.float32)
    return d1 + (d2 + d3)


def _body(lg_ref, box_ref, ts_ref, pm_ref, sc_ref, lb_ref, bx_ref):
    ys, kbs = [], []
    for i in range(IMGS):
        x = lg_ref[i]
        s = jax.nn.sigmoid(x)
        y = lax.dot_general(s, pm_ref[...], (((1,), (0,)), ((), ())),
                            preferred_element_type=jnp.float32)
        ys.append(y)
        kbs.append(lax.bitcast_convert_type(y, jnp.int32))

    def bs(_, carry):
        out = []
        for i in range(IMGS):
            lo, hi = carry[2 * i], carry[2 * i + 1]
            mid = (lo + hi) // 2
            cnt = jnp.sum((kbs[i] >= mid).astype(jnp.int32))
            big = cnt >= NSEL
            out += [jnp.where(big, mid, lo), jnp.where(big, hi, mid)]
        return tuple(out)

    init = (jnp.int32(1), jnp.int32(1 << 30)) * IMGS
    res = lax.fori_loop(0, 30, bs, init)
    for i in range(IMGS):
        _post(i, ys[i], kbs[i], res[2 * i],
              box_ref, ts_ref, sc_ref, lb_ref, bx_ref)


def _post(i, y, kbits, tstar, box_ref, ts_ref, sc_ref, lb_ref, bx_ref):
    mask = (kbits >= tstar).astype(jnp.int32)

    rc = jnp.sum(mask, axis=1, keepdims=True)
    cum = rc
    sh = 1
    while sh < _R:
        cum = cum + jnp.concatenate(
            [jnp.zeros((sh, 1), jnp.int32), cum[:-sh, :]], axis=0)
        sh *= 2
    total = cum[_R - 1, 0]

    cl = mask
    sh = 1
    while sh < _LANES:
        cl = cl + jnp.concatenate(
            [jnp.zeros((_R, sh), jnp.int32), cl[:, :-sh]], axis=1)
        sh *= 2

    p_col = lax.broadcasted_iota(jnp.int32, (CAND, 1), 0)
    cum_row = jnp.transpose(cum)
    a = (jnp.broadcast_to(cum_row, (CAND, _R))
         <= lax.broadcasted_iota(jnp.int32, (CAND, _R), 0)).astype(jnp.float32)
    ones_rc = jnp.concatenate(
        [jnp.ones((_R, 1), jnp.float32), rc.astype(jnp.float32)],
        axis=1)
    rr = lax.dot_general(a, ones_rc, (((1,), (0,)), ((), ())),
                         preferred_element_type=jnp.float32)
    r_p = rr[:, 0:1].astype(jnp.int32)
    ro_p = rr[:, 1:2].astype(jnp.int32)
    w_p = p_col - ro_p
    r_pc = jnp.minimum(r_p, _R - 1)
    onehot_r = (lax.broadcasted_iota(jnp.int32, (CAND, _R), 1)
                == r_pc).astype(jnp.float32)
    clg = lax.dot_general(onehot_r, cl.astype(jnp.float32),
                          (((1,), (0,)), ((), ())),
                          preferred_element_type=jnp.float32)

    bf = box_ref[i]
    fwd2 = jnp.concatenate([bf[:, -2:], bf[:, :-2]], axis=1)
    bwd2 = jnp.concatenate([bf[:, 2:], bf[:, :2]], axis=1)
    lmod = lax.broadcasted_iota(jnp.int32, (_R, 16), 1) % 4
    xyxy_fold = jnp.where(lmod < 2, bf - 0.5 * bwd2, fwd2 + 0.5 * bf)

    vb = jnp.concatenate([y, xyxy_fold], axis=1)
    ygb = _dot_exact(onehot_r, vb)
    yg = ygb[:, :_LANES]
    bg16 = ygb[:, _LANES:_LANES + 16]
    lcmp = (clg <= w_p.astype(jnp.float32)).astype(jnp.float32)
    l_p = lax.dot_general(lcmp, jnp.ones((_LANES, 1), jnp.float32),
                          (((1,), (0,)), ((), ())),
                          preferred_element_type=jnp.float32).astype(jnp.int32)
    l_pc = jnp.minimum(l_p, _LANES - 1)
    lmask = (lax.broadcasted_iota(jnp.int32, (CAND, _LANES), 1) == l_pc)
    val = jnp.sum(jnp.where(lmask, yg, 0.0), axis=1, keepdims=True)

    flat = r_pc * _M + l_pc
    valid = p_col < total
    key = lax.bitcast_convert_type(val, jnp.int32)
    key = jnp.where(valid, key, -1)
    flat_tb = jnp.where(valid, flat, (1 << 29) + p_col)
    q_p = flat // _C
    c_p = flat - q_p * _C

    key_row = jnp.transpose(key)
    flat_row = jnp.transpose(flat_tb)
    prec = ((key_row > key) |
            ((key_row == key) & (flat_row < flat_tb))).astype(jnp.float32)
    rank = lax.dot_general(prec, jnp.ones((CAND, 1), jnp.float32),
                           (((1,), (0,)), ((), ())),
                           preferred_element_type=jnp.float32).astype(jnp.int32)

    rank_row = jnp.transpose(rank)
    j_col = lax.broadcasted_iota(jnp.int32, (NSEL, CAND), 0)
    perm = (jnp.broadcast_to(rank_row, (NSEL, CAND))
            == j_col).astype(jnp.float32)

    jj = l_pc // _C
    mask16 = (lax.broadcasted_iota(jnp.int32, (CAND, 16), 1) // 4 == jj)
    bsel = jnp.where(mask16, bg16, 0.0)
    bq = (bsel[:, 0:4] + bsel[:, 4:8]) + (bsel[:, 8:12] + bsel[:, 12:16])

    fields = jnp.concatenate([val, c_p.astype(jnp.float32), bq], axis=1)
    sorted_f = _dot_exact(perm, fields)
    sc_ref[i] = sorted_f[:, 0:1]
    lb_ref[i] = sorted_f[:, 1:2].astype(jnp.int32)

    img_h = ts_ref[i, 0, 0].astype(jnp.float32)
    img_w = ts_ref[i, 0, 1].astype(jnp.float32)
    li = lax.broadcasted_iota(jnp.int32, (1, 4), 1)
    scale = jnp.where(li % 2 == 0, img_w, img_h)
    bx_ref[i] = sorted_f[:, 2:6] * scale


@jax.jit
def kernel(pred_logits, pred_boxes, target_sizes, positive_map):
    lg2 = pred_logits.reshape(_B, _R, _KIN)
    boxes_fold = pred_boxes.reshape(_B, _R, 16)
    pm_t = positive_map.T
    pm_big = jnp.zeros((_KIN, _LANES), jnp.float32)
    for j in range(_F):
        pm_big = pm_big.at[j * _T:(j + 1) * _T, j * _C:(j + 1) * _C].set(pm_t)
    ts3 = target_sizes.reshape(_B, 1, 2)

    grid = (_B // IMGS,)
    scores3, labels3, boxes = pl.pallas_call(
        _body,
        grid=grid,
        in_specs=[
            pl.BlockSpec((IMGS, _R, _KIN), lambda b: (b, 0, 0)),
            pl.BlockSpec((IMGS, _R, 16), lambda b: (b, 0, 0)),
            pl.BlockSpec((IMGS, 1, 2), lambda b: (b, 0, 0),
                         memory_space=pltpu.SMEM),
            pl.BlockSpec((_KIN, _LANES), lambda b: (0, 0)),
        ],
        out_specs=[
            pl.BlockSpec((IMGS, NSEL, 1), lambda b: (b, 0, 0)),
            pl.BlockSpec((IMGS, NSEL, 1), lambda b: (b, 0, 0)),
            pl.BlockSpec((IMGS, NSEL, 4), lambda b: (b, 0, 0)),
        ],
        out_shape=[
            jax.ShapeDtypeStruct((_B, NSEL, 1), jnp.float32),
            jax.ShapeDtypeStruct((_B, NSEL, 1), jnp.int32),
            jax.ShapeDtypeStruct((_B, NSEL, 4), jnp.float32),
        ],
    )(lg2, boxes_fold, ts3, pm_big)
    return scores3[..., 0], labels3[..., 0], boxes


# speedup vs baseline: 2.9423x; 1.2801x over previous
import functools

import jax
import jax.numpy as jnp
from jax import lax
from jax.experimental import pallas as pl
from jax.experimental.pallas import tpu as pltpu

NSEL = 300
CAND = 384
IMGS = 2
_B, _Q, _T, _C = 128, 900, 256, 25
_F = 4
_R = _Q // _F
_M = _F * _C
_LANES = 128
_KIN = _F * _T


def _split3(v):
    m = jnp.int32(-65536)
    b1 = lax.bitcast_convert_type(
        lax.bitcast_convert_type(v, jnp.int32) & m, jnp.float32)
    r1 = v - b1
    b2 = lax.bitcast_convert_type(
        lax.bitcast_convert_type(r1, jnp.int32) & m, jnp.float32)
    b3 = r1 - b2
    return b1, b2, b3


def _dot_exact(a, v):
    b1, b2, b3 = _split3(v)
    dn = (((1,), (0,)), ((), ()))
    d1 = lax.dot_general(a, b1, dn, preferred_element_type=jnp.float32)
    d2 = lax.dot_general(a, b2, dn, preferred_element_type=jnp.float32)
    d3 = lax.dot_general(a, b3, dn, preferred_element_type=jnp.float32)
    return d1 + (d2 + d3)


def _body(lg_ref, box_ref, ts_ref, pm_ref, sc_ref, lb_ref, bx_ref):
    ys, kbs = [], []
    for i in range(IMGS):
        x = lg_ref[i]
        s = jax.nn.sigmoid(x)
        y_un = lax.dot_general(s, pm_ref[...], (((1,), (0,)), ((), ())),
                               preferred_element_type=jnp.float32)
        y = jnp.concatenate(
            [y_un[j * _R:(j + 1) * _R, :_C] for j in range(_F)]
            + [jnp.zeros((_R, _LANES - _M), jnp.float32)], axis=1)
        ys.append(y)
        kbs.append(lax.bitcast_convert_type(y, jnp.int32))

    def bs(_, carry):
        out = []
        for i in range(IMGS):
            lo, hi = carry[2 * i], carry[2 * i + 1]
            mid = (lo + hi) // 2
            cnt = jnp.sum((kbs[i] >= mid).astype(jnp.int32))
            big = cnt >= NSEL
            out += [jnp.where(big, mid, lo), jnp.where(big, hi, mid)]
        return tuple(out)

    init = (jnp.int32(1), jnp.int32(1 << 30)) * IMGS
    res = lax.fori_loop(0, 30, bs, init)
    for i in range(IMGS):
        _post(i, ys[i], kbs[i], res[2 * i],
              box_ref, ts_ref, sc_ref, lb_ref, bx_ref)


def _post(i, y, kbits, tstar, box_ref, ts_ref, sc_ref, lb_ref, bx_ref):
    mask = (kbits >= tstar).astype(jnp.int32)

    rc = jnp.sum(mask, axis=1, keepdims=True)
    cum = rc
    sh = 1
    while sh < _R:
        cum = cum + jnp.concatenate(
            [jnp.zeros((sh, 1), jnp.int32), cum[:-sh, :]], axis=0)
        sh *= 2
    total = cum[_R - 1, 0]

    cl = mask
    sh = 1
    while sh < _LANES:
        cl = cl + jnp.concatenate(
            [jnp.zeros((_R, sh), jnp.int32), cl[:, :-sh]], axis=1)
        sh *= 2

    p_col = lax.broadcasted_iota(jnp.int32, (CAND, 1), 0)
    cum_row = jnp.transpose(cum)
    a = (jnp.broadcast_to(cum_row, (CAND, _R))
         <= lax.broadcasted_iota(jnp.int32, (CAND, _R), 0)).astype(jnp.float32)
    ones_rc = jnp.concatenate(
        [jnp.ones((_R, 1), jnp.float32), rc.astype(jnp.float32)],
        axis=1)
    rr = lax.dot_general(a, ones_rc, (((1,), (0,)), ((), ())),
                         preferred_element_type=jnp.float32)
    r_p = rr[:, 0:1].astype(jnp.int32)
    ro_p = rr[:, 1:2].astype(jnp.int32)
    w_p = p_col - ro_p
    r_pc = jnp.minimum(r_p, _R - 1)
    onehot_r = (lax.broadcasted_iota(jnp.int32, (CAND, _R), 1)
                == r_pc).astype(jnp.float32)
    clg = lax.dot_general(onehot_r, cl.astype(jnp.float32),
                          (((1,), (0,)), ((), ())),
                          preferred_element_type=jnp.float32)

    bx = box_ref[i]
    cx = bx[:, 0:1]
    cy = bx[:, 1:2]
    w = bx[:, 2:3]
    h = bx[:, 3:4]
    xyxy_un = jnp.concatenate(
        [cx - 0.5 * w, cy - 0.5 * h, cx + 0.5 * w, cy + 0.5 * h], axis=1)
    xyxy_fold = jnp.concatenate(
        [xyxy_un[j * _R:(j + 1) * _R, :] for j in range(_F)], axis=1)

    vb = jnp.concatenate([y, xyxy_fold], axis=1)
    ygb = _dot_exact(onehot_r, vb)
    yg = ygb[:, :_LANES]
    bg16 = ygb[:, _LANES:_LANES + 16]
    lcmp = (clg <= w_p.astype(jnp.float32)).astype(jnp.float32)
    l_p = lax.dot_general(lcmp, jnp.ones((_LANES, 1), jnp.float32),
                          (((1,), (0,)), ((), ())),
                          preferred_element_type=jnp.float32).astype(jnp.int32)
    l_pc = jnp.minimum(l_p, _LANES - 1)
    lmask = (lax.broadcasted_iota(jnp.int32, (CAND, _LANES), 1) == l_pc)
    val = jnp.sum(jnp.where(lmask, yg, 0.0), axis=1, keepdims=True)

    jp = l_pc // _C
    c_p = l_pc - jp * _C
    q_p = jp * _R + r_pc
    flat = q_p * _C + c_p
    valid = p_col < total
    key = lax.bitcast_convert_type(val, jnp.int32)
    key = jnp.where(valid, key, -1)
    flat_tb = jnp.where(valid, flat, (1 << 29) + p_col)

    key_row = jnp.transpose(key)
    flat_row = jnp.transpose(flat_tb)
    prec = ((key_row > key) |
            ((key_row == key) & (flat_row < flat_tb))).astype(jnp.float32)
    rank = lax.dot_general(prec, jnp.ones((CAND, 1), jnp.float32),
                           (((1,), (0,)), ((), ())),
                           preferred_element_type=jnp.float32).astype(jnp.int32)

    rank_row = jnp.transpose(rank)
    j_col = lax.broadcasted_iota(jnp.int32, (NSEL, CAND), 0)
    perm = (jnp.broadcast_to(rank_row, (NSEL, CAND))
            == j_col).astype(jnp.float32)

    mask16 = (lax.broadcasted_iota(jnp.int32, (CAND, 16), 1) // 4 == jp)
    bsel = jnp.where(mask16, bg16, 0.0)
    bq = (bsel[:, 0:4] + bsel[:, 4:8]) + (bsel[:, 8:12] + bsel[:, 12:16])

    fields = jnp.concatenate([val, c_p.astype(jnp.float32), bq], axis=1)
    sorted_f = _dot_exact(perm, fields)
    sc_ref[i] = sorted_f[:, 0:1]
    lb_ref[i] = sorted_f[:, 1:2].astype(jnp.int32)

    img_h = ts_ref[i, 0, 0].astype(jnp.float32)
    img_w = ts_ref[i, 0, 1].astype(jnp.float32)
    li = lax.broadcasted_iota(jnp.int32, (1, 4), 1)
    scale = jnp.where(li % 2 == 0, img_w, img_h)
    bx_ref[i] = sorted_f[:, 2:6] * scale


@jax.jit
def kernel(pred_logits, pred_boxes, target_sizes, positive_map):
    pm_pad = jnp.zeros((_T, _LANES), jnp.float32).at[:, :_C].set(
        positive_map.T)
    ts3 = target_sizes.reshape(_B, 1, 2)

    grid = (_B // IMGS,)
    scores3, labels3, boxes = pl.pallas_call(
        _body,
        grid=grid,
        in_specs=[
            pl.BlockSpec((IMGS, _Q, _T), lambda b: (b, 0, 0)),
            pl.BlockSpec((IMGS, _Q, 4), lambda b: (b, 0, 0)),
            pl.BlockSpec((IMGS, 1, 2), lambda b: (b, 0, 0),
                         memory_space=pltpu.SMEM),
            pl.BlockSpec((_T, _LANES), lambda b: (0, 0)),
        ],
        out_specs=[
            pl.BlockSpec((IMGS, NSEL, 1), lambda b: (b, 0, 0)),
            pl.BlockSpec((IMGS, NSEL, 1), lambda b: (b, 0, 0)),
            pl.BlockSpec((IMGS, NSEL, 4), lambda b: (b, 0, 0)),
        ],
        out_shape=[
            jax.ShapeDtypeStruct((_B, NSEL, 1), jnp.float32),
            jax.ShapeDtypeStruct((_B, NSEL, 1), jnp.int32),
            jax.ShapeDtypeStruct((_B, NSEL, 4), jnp.float32),
        ],
    )(pred_logits, pred_boxes, ts3, pm_pad)
    return scores3[..., 0], labels3[..., 0], boxes


# speedup vs baseline: 2.9444x; 1.0007x over previous
import functools

import jax
import jax.numpy as jnp
from jax import lax
from jax.experimental import pallas as pl
from jax.experimental.pallas import tpu as pltpu

NSEL = 300
CAND = 384
IMGS = 4
_B, _Q, _T, _C = 128, 900, 256, 25
_F = 4
_R = _Q // _F
_M = _F * _C
_LANES = 128
_KIN = _F * _T


def _split3(v):
    m = jnp.int32(-65536)
    b1 = lax.bitcast_convert_type(
        lax.bitcast_convert_type(v, jnp.int32) & m, jnp.float32)
    r1 = v - b1
    b2 = lax.bitcast_convert_type(
        lax.bitcast_convert_type(r1, jnp.int32) & m, jnp.float32)
    b3 = r1 - b2
    return b1, b2, b3


def _dot_exact(a, v):
    b1, b2, b3 = _split3(v)
    dn = (((1,), (0,)), ((), ()))
    d1 = lax.dot_general(a, b1, dn, preferred_element_type=jnp.float32)
    d2 = lax.dot_general(a, b2, dn, preferred_element_type=jnp.float32)
    d3 = lax.dot_general(a, b3, dn, preferred_element_type=jnp.float32)
    return d1 + (d2 + d3)


def _body(lg_ref, box_ref, ts_ref, pm_ref, sc_ref, lb_ref, bx_ref):
    ys, kbs = [], []
    for i in range(IMGS):
        x = lg_ref[i]
        s = jax.nn.sigmoid(x)
        y_un = lax.dot_general(s, pm_ref[...], (((1,), (0,)), ((), ())),
                               preferred_element_type=jnp.float32)
        y = jnp.concatenate(
            [y_un[j * _R:(j + 1) * _R, :_C] for j in range(_F)]
            + [jnp.zeros((_R, _LANES - _M), jnp.float32)], axis=1)
        ys.append(y)
        kbs.append(lax.bitcast_convert_type(y, jnp.int32))

    def bs(_, carry):
        out = []
        for i in range(IMGS):
            lo, hi = carry[2 * i], carry[2 * i + 1]
            mid = (lo + hi) // 2
            cnt = jnp.sum((kbs[i] >= mid).astype(jnp.int32))
            big = cnt >= NSEL
            out += [jnp.where(big, mid, lo), jnp.where(big, hi, mid)]
        return tuple(out)

    init = (jnp.int32(1), jnp.int32(1 << 30)) * IMGS
    res = lax.fori_loop(0, 30, bs, init)
    for i in range(IMGS):
        _post(i, ys[i], kbs[i], res[2 * i],
              box_ref, ts_ref, sc_ref, lb_ref, bx_ref)


def _post(i, y, kbits, tstar, box_ref, ts_ref, sc_ref, lb_ref, bx_ref):
    mask = (kbits >= tstar).astype(jnp.int32)

    rc = jnp.sum(mask, axis=1, keepdims=True)
    cum = rc
    sh = 1
    while sh < _R:
        cum = cum + jnp.concatenate(
            [jnp.zeros((sh, 1), jnp.int32), cum[:-sh, :]], axis=0)
        sh *= 2
    total = cum[_R - 1, 0]

    li = lax.broadcasted_iota(jnp.int32, (_LANES, _LANES), 0)
    lj = lax.broadcasted_iota(jnp.int32, (_LANES, _LANES), 1)
    tri = (li <= lj).astype(jnp.float32)
    cl = lax.dot_general(mask.astype(jnp.float32), tri,
                         (((1,), (0,)), ((), ())),
                         preferred_element_type=jnp.float32)

    p_col = lax.broadcasted_iota(jnp.int32, (CAND, 1), 0)
    cum_row = jnp.transpose(cum)
    a = (jnp.broadcast_to(cum_row, (CAND, _R))
         <= lax.broadcasted_iota(jnp.int32, (CAND, _R), 0)).astype(jnp.float32)
    ones_rc = jnp.concatenate(
        [jnp.ones((_R, 1), jnp.float32), rc.astype(jnp.float32)],
        axis=1)
    rr = lax.dot_general(a, ones_rc, (((1,), (0,)), ((), ())),
                         preferred_element_type=jnp.float32)
    r_p = rr[:, 0:1].astype(jnp.int32)
    ro_p = rr[:, 1:2].astype(jnp.int32)
    w_p = p_col - ro_p
    r_pc = jnp.minimum(r_p, _R - 1)
    onehot_r = (lax.broadcasted_iota(jnp.int32, (CAND, _R), 1)
                == r_pc).astype(jnp.float32)
    clg = lax.dot_general(onehot_r, cl,
                          (((1,), (0,)), ((), ())),
                          preferred_element_type=jnp.float32)

    bx = box_ref[i]
    cx = bx[:, 0:1]
    cy = bx[:, 1:2]
    w = bx[:, 2:3]
    h = bx[:, 3:4]
    xyxy_un = jnp.concatenate(
        [cx - 0.5 * w, cy - 0.5 * h, cx + 0.5 * w, cy + 0.5 * h], axis=1)
    xyxy_fold = jnp.concatenate(
        [xyxy_un[j * _R:(j + 1) * _R, :] for j in range(_F)], axis=1)

    vb = jnp.concatenate([y, xyxy_fold], axis=1)
    ygb = _dot_exact(onehot_r, vb)
    yg = ygb[:, :_LANES]
    bg16 = ygb[:, _LANES:_LANES + 16]
    lcmp = (clg <= w_p.astype(jnp.float32)).astype(jnp.float32)
    l_p = lax.dot_general(lcmp, jnp.ones((_LANES, 1), jnp.float32),
                          (((1,), (0,)), ((), ())),
                          preferred_element_type=jnp.float32).astype(jnp.int32)
    l_pc = jnp.minimum(l_p, _LANES - 1)
    lmask = (lax.broadcasted_iota(jnp.int32, (CAND, _LANES), 1) == l_pc)
    val = jnp.sum(jnp.where(lmask, yg, 0.0), axis=1, keepdims=True)

    jp = l_pc // _C
    c_p = l_pc - jp * _C
    q_p = jp * _R + r_pc
    flat = q_p * _C + c_p
    valid = p_col < total
    key = lax.bitcast_convert_type(val, jnp.int32)
    key = jnp.where(valid, key, -1)
    flat_tb = jnp.where(valid, flat, (1 << 29) + p_col)

    key_row = jnp.transpose(key)
    flat_row = jnp.transpose(flat_tb)
    prec = ((key_row > key) |
            ((key_row == key) & (flat_row < flat_tb))).astype(jnp.float32)
    rank = lax.dot_general(prec, jnp.ones((CAND, 1), jnp.float32),
                           (((1,), (0,)), ((), ())),
                           preferred_element_type=jnp.float32).astype(jnp.int32)

    rank_row = jnp.transpose(rank)
    j_col = lax.broadcasted_iota(jnp.int32, (NSEL, CAND), 0)
    perm = (jnp.broadcast_to(rank_row, (NSEL, CAND))
            == j_col).astype(jnp.float32)

    mask16 = (lax.broadcasted_iota(jnp.int32, (CAND, 16), 1) // 4 == jp)
    bsel = jnp.where(mask16, bg16, 0.0)
    bq = (bsel[:, 0:4] + bsel[:, 4:8]) + (bsel[:, 8:12] + bsel[:, 12:16])

    fields = jnp.concatenate([val, c_p.astype(jnp.float32), bq], axis=1)
    sorted_f = _dot_exact(perm, fields)
    sc_ref[i] = sorted_f[:, 0:1]
    lb_ref[i] = sorted_f[:, 1:2].astype(jnp.int32)

    img_h = ts_ref[i, 0, 0].astype(jnp.float32)
    img_w = ts_ref[i, 0, 1].astype(jnp.float32)
    li = lax.broadcasted_iota(jnp.int32, (1, 4), 1)
    scale = jnp.where(li % 2 == 0, img_w, img_h)
    bx_ref[i] = sorted_f[:, 2:6] * scale


@jax.jit
def kernel(pred_logits, pred_boxes, target_sizes, positive_map):
    pm_pad = jnp.zeros((_T, _LANES), jnp.float32).at[:, :_C].set(
        positive_map.T)
    ts3 = target_sizes.reshape(_B, 1, 2)

    grid = (_B // IMGS,)
    scores3, labels3, boxes = pl.pallas_call(
        _body,
        grid=grid,
        in_specs=[
            pl.BlockSpec((IMGS, _Q, _T), lambda b: (b, 0, 0)),
            pl.BlockSpec((IMGS, _Q, 4), lambda b: (b, 0, 0)),
            pl.BlockSpec((IMGS, 1, 2), lambda b: (b, 0, 0),
                         memory_space=pltpu.SMEM),
            pl.BlockSpec((_T, _LANES), lambda b: (0, 0)),
        ],
        out_specs=[
            pl.BlockSpec((IMGS, NSEL, 1), lambda b: (b, 0, 0)),
            pl.BlockSpec((IMGS, NSEL, 1), lambda b: (b, 0, 0)),
            pl.BlockSpec((IMGS, NSEL, 4), lambda b: (b, 0, 0)),
        ],
        out_shape=[
            jax.ShapeDtypeStruct((_B, NSEL, 1), jnp.float32),
            jax.ShapeDtypeStruct((_B, NSEL, 1), jnp.int32),
            jax.ShapeDtypeStruct((_B, NSEL, 4), jnp.float32),
        ],
    )(pred_logits, pred_boxes, ts3, pm_pad)
    return scores3[..., 0], labels3[..., 0], boxes


# speedup vs baseline: 3.2056x; 1.0887x over previous
import jax
import jax.numpy as jnp
from jax import lax
from jax.experimental import pallas as pl
from jax.experimental.pallas import tpu as pltpu

NSEL = 300
CAND = 384
IMGS = 8
_B, _Q, _T, _C = 128, 900, 256, 25
_F = 4
_R = _Q // _F
_M = _F * _C
_LANES = 128
_KIN = _F * _T


def _split3(v):
    m = jnp.int32(-65536)
    b1 = lax.bitcast_convert_type(
        lax.bitcast_convert_type(v, jnp.int32) & m, jnp.float32)
    r1 = v - b1
    b2 = lax.bitcast_convert_type(
        lax.bitcast_convert_type(r1, jnp.int32) & m, jnp.float32)
    b3 = r1 - b2
    return b1, b2, b3


def _dot_exact(a, v):
    b1, b2, b3 = _split3(v)
    dn = (((1,), (0,)), ((), ()))
    d1 = lax.dot_general(a, b1, dn, preferred_element_type=jnp.float32)
    d2 = lax.dot_general(a, b2, dn, preferred_element_type=jnp.float32)
    d3 = lax.dot_general(a, b3, dn, preferred_element_type=jnp.float32)
    return d1 + (d2 + d3)


def _body(lg_ref, box_ref, ts_ref, pm_ref, sc_ref, lb_ref, bx_ref):
    ys, kbs = [], []
    for i in range(IMGS):
        x = lg_ref[i]
        s = jax.nn.sigmoid(x)
        y_un = lax.dot_general(s, pm_ref[...], (((1,), (0,)), ((), ())),
                               preferred_element_type=jnp.float32)
        y = jnp.concatenate(
            [y_un[j * _R:(j + 1) * _R, :_C] for j in range(_F)]
            + [jnp.zeros((_R, _LANES - _M), jnp.float32)], axis=1)
        ys.append(y)
        kbs.append(lax.bitcast_convert_type(y, jnp.int32))

    def bs(_, carry):
        out = []
        for i in range(IMGS):
            lo, hi = carry[2 * i], carry[2 * i + 1]
            mid = (lo + hi) // 2
            cnt = jnp.sum((kbs[i] >= mid).astype(jnp.int32))
            big = cnt >= NSEL
            out += [jnp.where(big, mid, lo), jnp.where(big, hi, mid)]
        return tuple(out)

    init = (jnp.int32(1), jnp.int32(1 << 30)) * IMGS
    res = lax.fori_loop(0, 30, bs, init)
    for i in range(IMGS):
        _post(i, ys[i], kbs[i], res[2 * i],
              box_ref, ts_ref, sc_ref, lb_ref, bx_ref)


def _post(i, y, kbits, tstar, box_ref, ts_ref, sc_ref, lb_ref, bx_ref):
    mask = (kbits >= tstar).astype(jnp.int32)

    rc = jnp.sum(mask, axis=1, keepdims=True)
    cum = rc
    sh = 1
    while sh < _R:
        cum = cum + jnp.concatenate(
            [jnp.zeros((sh, 1), jnp.int32), cum[:-sh, :]], axis=0)
        sh *= 2
    total = cum[_R - 1, 0]

    li = lax.broadcasted_iota(jnp.int32, (_LANES, _LANES), 0)
    lj = lax.broadcasted_iota(jnp.int32, (_LANES, _LANES), 1)
    tri = (li <= lj).astype(jnp.float32)
    cl = lax.dot_general(mask.astype(jnp.float32), tri,
                         (((1,), (0,)), ((), ())),
                         preferred_element_type=jnp.float32)

    p_col = lax.broadcasted_iota(jnp.int32, (CAND, 1), 0)
    cum_row = jnp.transpose(cum)
    a = (jnp.broadcast_to(cum_row, (CAND, _R))
         <= lax.broadcasted_iota(jnp.int32, (CAND, _R), 0)).astype(jnp.float32)
    ones_rc = jnp.concatenate(
        [jnp.ones((_R, 1), jnp.float32), rc.astype(jnp.float32)],
        axis=1)
    rr = lax.dot_general(a, ones_rc, (((1,), (0,)), ((), ())),
                         preferred_element_type=jnp.float32)
    r_p = rr[:, 0:1].astype(jnp.int32)
    ro_p = rr[:, 1:2].astype(jnp.int32)
    w_p = p_col - ro_p
    r_pc = jnp.minimum(r_p, _R - 1)
    onehot_r = (lax.broadcasted_iota(jnp.int32, (CAND, _R), 1)
                == r_pc).astype(jnp.float32)
    clg = lax.dot_general(onehot_r, cl,
                          (((1,), (0,)), ((), ())),
                          preferred_element_type=jnp.float32)

    bx = box_ref[i]
    cx = bx[:, 0:1]
    cy = bx[:, 1:2]
    w = bx[:, 2:3]
    h = bx[:, 3:4]
    xyxy_un = jnp.concatenate(
        [cx - 0.5 * w, cy - 0.5 * h, cx + 0.5 * w, cy + 0.5 * h], axis=1)
    xyxy_fold = jnp.concatenate(
        [xyxy_un[j * _R:(j + 1) * _R, :] for j in range(_F)], axis=1)

    vb = jnp.concatenate([y, xyxy_fold], axis=1)
    ygb = _dot_exact(onehot_r, vb)
    yg = ygb[:, :_LANES]
    bg16 = ygb[:, _LANES:_LANES + 16]
    lcmp = (clg <= w_p.astype(jnp.float32)).astype(jnp.float32)
    l_p = lax.dot_general(lcmp, jnp.ones((_LANES, 1), jnp.float32),
                          (((1,), (0,)), ((), ())),
                          preferred_element_type=jnp.float32).astype(jnp.int32)
    l_pc = jnp.minimum(l_p, _LANES - 1)
    lmask = (lax.broadcasted_iota(jnp.int32, (CAND, _LANES), 1) == l_pc)
    val = jnp.sum(jnp.where(lmask, yg, 0.0), axis=1, keepdims=True)

    jp = l_pc // _C
    c_p = l_pc - jp * _C
    q_p = jp * _R + r_pc
    flat = q_p * _C + c_p
    valid = p_col < total
    key = lax.bitcast_convert_type(val, jnp.int32)
    key = jnp.where(valid, key, -1)
    flat_tb = jnp.where(valid, flat, (1 << 29) + p_col)

    key_row = jnp.transpose(key)
    flat_row = jnp.transpose(flat_tb)
    prec = ((key_row > key) |
            ((key_row == key) & (flat_row < flat_tb))).astype(jnp.float32)
    rank = lax.dot_general(prec, jnp.ones((CAND, 1), jnp.float32),
                           (((1,), (0,)), ((), ())),
                           preferred_element_type=jnp.float32).astype(jnp.int32)

    rank_row = jnp.transpose(rank)
    j_col = lax.broadcasted_iota(jnp.int32, (NSEL, CAND), 0)
    perm = (jnp.broadcast_to(rank_row, (NSEL, CAND))
            == j_col).astype(jnp.float32)

    mask16 = (lax.broadcasted_iota(jnp.int32, (CAND, 16), 1) // 4 == jp)
    bsel = jnp.where(mask16, bg16, 0.0)
    bq = (bsel[:, 0:4] + bsel[:, 4:8]) + (bsel[:, 8:12] + bsel[:, 12:16])

    fields = jnp.concatenate([val, c_p.astype(jnp.float32), bq], axis=1)
    sorted_f = _dot_exact(perm, fields)
    sc_ref[i] = jnp.reshape(jnp.transpose(sorted_f[:, 0:1]), (NSEL,))
    lb_ref[i] = jnp.reshape(jnp.transpose(sorted_f[:, 1:2]), (NSEL,)).astype(jnp.int32)

    img_h = ts_ref[i, 0, 0].astype(jnp.float32)
    img_w = ts_ref[i, 0, 1].astype(jnp.float32)
    li = lax.broadcasted_iota(jnp.int32, (1, 4), 1)
    scale = jnp.where(li % 2 == 0, img_w, img_h)
    bx_ref[i] = sorted_f[:, 2:6] * scale


@jax.jit
def kernel(pred_logits, pred_boxes, target_sizes, positive_map):
    pm_pad = jnp.zeros((_T, _LANES), jnp.float32).at[:, :_C].set(
        positive_map.T)
    ts3 = target_sizes.reshape(_B, 1, 2)

    grid = (_B // IMGS,)
    scores, labels, boxes = pl.pallas_call(
        _body,
        grid=grid,
        in_specs=[
            pl.BlockSpec((IMGS, _Q, _T), lambda b: (b, 0, 0)),
            pl.BlockSpec((IMGS, _Q, 4), lambda b: (b, 0, 0)),
            pl.BlockSpec((IMGS, 1, 2), lambda b: (b, 0, 0),
                         memory_space=pltpu.SMEM),
            pl.BlockSpec((_T, _LANES), lambda b: (0, 0)),
        ],
        out_specs=[
            pl.BlockSpec((IMGS, NSEL), lambda b: (b, 0)),
            pl.BlockSpec((IMGS, NSEL), lambda b: (b, 0)),
            pl.BlockSpec((IMGS, NSEL, 4), lambda b: (b, 0, 0)),
        ],
        out_shape=[
            jax.ShapeDtypeStruct((_B, NSEL), jnp.float32),
            jax.ShapeDtypeStruct((_B, NSEL), jnp.int32),
            jax.ShapeDtypeStruct((_B, NSEL, 4), jnp.float32),
        ],
    )(pred_logits, pred_boxes, ts3, pm_pad)
    return scores, labels, boxes


# speedup vs baseline: 3.5390x; 1.1040x over previous
import jax
import jax.numpy as jnp
from jax import lax
from jax.experimental import pallas as pl
from jax.experimental.pallas import tpu as pltpu

NSEL = 300
CAND = 384
IMGS = 8
_B, _Q, _T, _C = 128, 900, 256, 25
_F = 4
_R = _Q // _F
_M = _F * _C
_LANES = 128


def _split3(v):
    m = jnp.int32(-65536)
    b1 = lax.bitcast_convert_type(
        lax.bitcast_convert_type(v, jnp.int32) & m, jnp.float32)
    r1 = v - b1
    b2 = lax.bitcast_convert_type(
        lax.bitcast_convert_type(r1, jnp.int32) & m, jnp.float32)
    b3 = r1 - b2
    return b1, b2, b3


def _dot_exact(a, v):
    b1, b2, b3 = _split3(v)
    dn = (((1,), (0,)), ((), ()))
    d1 = lax.dot_general(a, b1, dn, preferred_element_type=jnp.float32)
    d2 = lax.dot_general(a, b2, dn, preferred_element_type=jnp.float32)
    d3 = lax.dot_general(a, b3, dn, preferred_element_type=jnp.float32)
    return d1 + (d2 + d3)


def _body(lg_ref, box_ref, ts_ref, pm_ref, sc_ref, lb_ref, bx_ref):
    ys, kbs = [], []
    for i in range(IMGS):
        x = lg_ref[i]
        s = jax.nn.sigmoid(x)
        y_un = lax.dot_general(s, pm_ref[...], (((1,), (0,)), ((), ())),
                               preferred_element_type=jnp.float32)
        y = jnp.concatenate(
            [y_un[j * _R:(j + 1) * _R, :_C] for j in range(_F)]
            + [jnp.zeros((_R, _LANES - _M), jnp.float32)], axis=1)
        ys.append(y)
        kbs.append(lax.bitcast_convert_type(y, jnp.int32))

    def cond(carry):
        alive = None
        for i in range(IMGS):
            lo, hi, cl_ = carry[3 * i], carry[3 * i + 1], carry[3 * i + 2]
            a = (hi - lo > 1) & (cl_ > CAND)
            alive = a if alive is None else (alive | a)
        return alive

    def bs(carry):
        out = []
        for i in range(IMGS):
            lo, hi, cl_ = carry[3 * i], carry[3 * i + 1], carry[3 * i + 2]
            act = (hi - lo > 1) & (cl_ > CAND)
            mid = (lo + hi) // 2
            cnt = jnp.sum((kbs[i] >= mid).astype(jnp.int32))
            big = cnt >= NSEL
            lo2 = jnp.where(act & big, mid, lo)
            hi2 = jnp.where(act & ~big, mid, hi)
            cl2 = jnp.where(act & big, cnt, cl_)
            out += [lo2, hi2, cl2]
        return tuple(out)

    init = (jnp.int32(1), jnp.int32(1 << 30), jnp.int32(_Q * _C)) * IMGS
    res = lax.while_loop(cond, bs, init)
    for i in range(IMGS):
        _post(i, ys[i], kbs[i], res[3 * i],
              box_ref, ts_ref, sc_ref, lb_ref, bx_ref)


def _post(i, y, kbits, tstar, box_ref, ts_ref, sc_ref, lb_ref, bx_ref):
    mask = (kbits >= tstar).astype(jnp.int32)

    rc = jnp.sum(mask, axis=1, keepdims=True)
    cum = rc
    sh = 1
    while sh < _R:
        cum = cum + jnp.concatenate(
            [jnp.zeros((sh, 1), jnp.int32), cum[:-sh, :]], axis=0)
        sh *= 2
    total = cum[_R - 1, 0]

    li = lax.broadcasted_iota(jnp.int32, (_LANES, _LANES), 0)
    lj = lax.broadcasted_iota(jnp.int32, (_LANES, _LANES), 1)
    tri = (li <= lj).astype(jnp.float32)
    cl = lax.dot_general(mask.astype(jnp.float32), tri,
                         (((1,), (0,)), ((), ())),
                         preferred_element_type=jnp.float32)

    p_col = lax.broadcasted_iota(jnp.int32, (CAND, 1), 0)
    cum_row = jnp.transpose(cum)
    a = (jnp.broadcast_to(cum_row, (CAND, _R))
         <= lax.broadcasted_iota(jnp.int32, (CAND, _R), 0)).astype(jnp.float32)
    ones_rc = jnp.concatenate(
        [jnp.ones((_R, 1), jnp.float32), rc.astype(jnp.float32)],
        axis=1)
    rr = lax.dot_general(a, ones_rc, (((1,), (0,)), ((), ())),
                         preferred_element_type=jnp.float32)
    r_p = rr[:, 0:1].astype(jnp.int32)
    ro_p = rr[:, 1:2].astype(jnp.int32)
    w_p = p_col - ro_p
    r_pc = jnp.minimum(r_p, _R - 1)
    onehot_r = (lax.broadcasted_iota(jnp.int32, (CAND, _R), 1)
                == r_pc).astype(jnp.float32)
    clg = lax.dot_general(onehot_r, cl,
                          (((1,), (0,)), ((), ())),
                          preferred_element_type=jnp.float32)

    bx = box_ref[i]
    cx = bx[:, 0:1]
    cy = bx[:, 1:2]
    w = bx[:, 2:3]
    h = bx[:, 3:4]
    xyxy_un = jnp.concatenate(
        [cx - 0.5 * w, cy - 0.5 * h, cx + 0.5 * w, cy + 0.5 * h], axis=1)
    xyxy_fold = jnp.concatenate(
        [xyxy_un[j * _R:(j + 1) * _R, :] for j in range(_F)], axis=1)

    vb = jnp.concatenate([y, xyxy_fold], axis=1)
    ygb = _dot_exact(onehot_r, vb)
    yg = ygb[:, :_LANES]
    bg16 = ygb[:, _LANES:_LANES + 16]
    lcmp = (clg <= w_p.astype(jnp.float32)).astype(jnp.float32)
    l_p = lax.dot_general(lcmp, jnp.ones((_LANES, 1), jnp.float32),
                          (((1,), (0,)), ((), ())),
                          preferred_element_type=jnp.float32).astype(jnp.int32)
    l_pc = jnp.minimum(l_p, _LANES - 1)
    lmask = (lax.broadcasted_iota(jnp.int32, (CAND, _LANES), 1) == l_pc)
    val = jnp.sum(jnp.where(lmask, yg, 0.0), axis=1, keepdims=True)

    jp = l_pc // _C
    c_p = l_pc - jp * _C
    q_p = jp * _R + r_pc
    flat = q_p * _C + c_p
    valid = p_col < total
    key = lax.bitcast_convert_type(val, jnp.int32)
    key = jnp.where(valid, key, -1)
    flat_tb = jnp.where(valid, flat, (1 << 29) + p_col)

    key_row = jnp.transpose(key)
    flat_row = jnp.transpose(flat_tb)
    prec = ((key_row > key) |
            ((key_row == key) & (flat_row < flat_tb))).astype(jnp.float32)
    rank = lax.dot_general(prec, jnp.ones((CAND, 1), jnp.float32),
                           (((1,), (0,)), ((), ())),
                           preferred_element_type=jnp.float32).astype(jnp.int32)

    rank_row = jnp.transpose(rank)
    j_col = lax.broadcasted_iota(jnp.int32, (NSEL, CAND), 0)
    perm = (jnp.broadcast_to(rank_row, (NSEL, CAND))
            == j_col).astype(jnp.float32)

    mask16 = (lax.broadcasted_iota(jnp.int32, (CAND, 16), 1) // 4 == jp)
    bsel = jnp.where(mask16, bg16, 0.0)
    bq = (bsel[:, 0:4] + bsel[:, 4:8]) + (bsel[:, 8:12] + bsel[:, 12:16])

    fields = jnp.concatenate([val, c_p.astype(jnp.float32), bq], axis=1)
    sorted_f = _dot_exact(perm, fields)
    sc_ref[i] = jnp.reshape(jnp.transpose(sorted_f[:, 0:1]), (NSEL,))
    lb_ref[i] = jnp.reshape(jnp.transpose(sorted_f[:, 1:2]), (NSEL,)).astype(jnp.int32)

    img_h = ts_ref[i, 0, 0].astype(jnp.float32)
    img_w = ts_ref[i, 0, 1].astype(jnp.float32)
    li = lax.broadcasted_iota(jnp.int32, (1, 4), 1)
    scale = jnp.where(li % 2 == 0, img_w, img_h)
    bx_ref[i] = sorted_f[:, 2:6] * scale


@jax.jit
def kernel(pred_logits, pred_boxes, target_sizes, positive_map):
    pm_pad = jnp.zeros((_T, _LANES), jnp.float32).at[:, :_C].set(
        positive_map.T)
    ts3 = target_sizes.reshape(_B, 1, 2)

    grid = (_B // IMGS,)
    scores, labels, boxes = pl.pallas_call(
        _body,
        grid=grid,
        in_specs=[
            pl.BlockSpec((IMGS, _Q, _T), lambda b: (b, 0, 0)),
            pl.BlockSpec((IMGS, _Q, 4), lambda b: (b, 0, 0)),
            pl.BlockSpec((IMGS, 1, 2), lambda b: (b, 0, 0),
                         memory_space=pltpu.SMEM),
            pl.BlockSpec((_T, _LANES), lambda b: (0, 0)),
        ],
        out_specs=[
            pl.BlockSpec((IMGS, NSEL), lambda b: (b, 0)),
            pl.BlockSpec((IMGS, NSEL), lambda b: (b, 0)),
            pl.BlockSpec((IMGS, NSEL, 4), lambda b: (b, 0, 0)),
        ],
        out_shape=[
            jax.ShapeDtypeStruct((_B, NSEL), jnp.float32),
            jax.ShapeDtypeStruct((_B, NSEL), jnp.int32),
            jax.ShapeDtypeStruct((_B, NSEL, 4), jnp.float32),
        ],
    )(pred_logits, pred_boxes, ts3, pm_pad)
    return scores, labels, boxes
